# initial kernel scaffold (unmeasured)
import jax
import jax.numpy as jnp
from jax import lax
from jax.experimental import pallas as pl
from jax.experimental.pallas import tpu as pltpu


def kernel(
    x,
):
    def body(*refs):
        pass

    out_shape = jax.ShapeDtypeStruct(..., jnp.float32)
    return pl.pallas_call(body, out_shape=out_shape)(...)


# baseline (device time: 235444 ns/iter reference)
import jax
import jax.numpy as jnp
from jax import lax
from jax.experimental import pallas as pl
from jax.experimental.pallas import tpu as pltpu

N_DEV = 16
M = 2048
N = 1024
CH = M // N_DEV
N_HOPS = 2 * (N_DEV - 1)


def kernel(x):
    def body(x_ref, out_ref, comm_ref, send_sems, recv_sems):
        me = lax.axis_index("i")
        left = (me - 1) % N_DEV
        right = (me + 1) % N_DEV

        barrier_sem = pltpu.get_barrier_semaphore()
        for nbr in (left, right):
            pl.semaphore_signal(
                barrier_sem, inc=1,
                device_id=(nbr,), device_id_type=pl.DeviceIdType.MESH,
            )
        pl.semaphore_wait(barrier_sem, 2)

        out_ref[:, :] = x_ref[0]

        for h in range(N_DEV - 1):
            c_send = (me - h) % N_DEV
            rdma = pltpu.make_async_remote_copy(
                src_ref=out_ref.at[pl.ds(c_send * CH, CH), :],
                dst_ref=comm_ref.at[h],
                send_sem=send_sems.at[h],
                recv_sem=recv_sems.at[h],
                device_id=(right,),
                device_id_type=pl.DeviceIdType.MESH,
            )
            rdma.start()
            rdma.wait()
            c_recv = (me - 1 - h) % N_DEV
            sl = pl.ds(c_recv * CH, CH)
            out_ref[sl, :] += comm_ref[h]

        for h in range(N_DEV - 1):
            c_send = (me + 1 - h) % N_DEV
            s = N_DEV - 1 + h
            rdma = pltpu.make_async_remote_copy(
                src_ref=out_ref.at[pl.ds(c_send * CH, CH), :],
                dst_ref=comm_ref.at[s],
                send_sem=send_sems.at[s],
                recv_sem=recv_sems.at[s],
                device_id=(right,),
                device_id_type=pl.DeviceIdType.MESH,
            )
            rdma.start()
            rdma.wait()
            c_recv = (me - h) % N_DEV
            out_ref[pl.ds(c_recv * CH, CH), :] = comm_ref[s]

    return pl.pallas_call(
        body,
        out_shape=jax.ShapeDtypeStruct((M, N), jnp.float32),
        in_specs=[pl.BlockSpec(memory_space=pltpu.VMEM)],
        out_specs=pl.BlockSpec(memory_space=pltpu.VMEM),
        scratch_shapes=[
            pltpu.VMEM((N_HOPS, CH, N), jnp.float32),
            pltpu.SemaphoreType.DMA((N_HOPS,)),
            pltpu.SemaphoreType.DMA((N_HOPS,)),
        ],
        compiler_params=pltpu.CompilerParams(collective_id=0),
    )(x)


# device time: 154194 ns/iter; 1.5269x vs baseline; 1.5269x over previous
import jax
import jax.numpy as jnp
from jax import lax
from jax.experimental import pallas as pl
from jax.experimental.pallas import tpu as pltpu

N_DEV = 16
M = 2048
N = 1024
CH = M // N_DEV
HALF = N // 2
N_HOPS = 2 * (N_DEV - 1)


def kernel(x):
    def body(x_ref, out_ref, comm_f, comm_b, sf_send, sf_recv, sb_send, sb_recv):
        me = lax.axis_index("i")
        left = (me - 1) % N_DEV
        right = (me + 1) % N_DEV

        def fwd_send(slot, src):
            rdma = pltpu.make_async_remote_copy(
                src_ref=src, dst_ref=comm_f.at[slot],
                send_sem=sf_send.at[slot], recv_sem=sf_recv.at[slot],
                device_id=(right,), device_id_type=pl.DeviceIdType.MESH,
            )
            rdma.start()

        def bwd_send(slot, src):
            rdma = pltpu.make_async_remote_copy(
                src_ref=src, dst_ref=comm_b.at[slot],
                send_sem=sb_send.at[slot], recv_sem=sb_recv.at[slot],
                device_id=(left,), device_id_type=pl.DeviceIdType.MESH,
            )
            rdma.start()

        def recv_wait(comm, recv_sems, slot):
            rdma = pltpu.make_async_remote_copy(
                src_ref=comm.at[slot], dst_ref=comm.at[slot],
                send_sem=sf_send.at[slot], recv_sem=recv_sems.at[slot],
                device_id=(right,), device_id_type=pl.DeviceIdType.MESH,
            )
            rdma.wait_recv()

        barrier_sem = pltpu.get_barrier_semaphore()
        for nbr in (left, right):
            pl.semaphore_signal(
                barrier_sem, inc=1,
                device_id=(nbr,), device_id_type=pl.DeviceIdType.MESH,
            )
        pl.semaphore_wait(barrier_sem, 2)

        out_ref[:, :] = x_ref[0]

        def f_rows(h):
            return pl.ds(((me - h) % N_DEV) * CH, CH)

        def b_rows(h):
            return pl.ds(((me + h) % N_DEV) * CH, CH)

        fwd_send(0, out_ref.at[f_rows(0), 0:HALF])
        bwd_send(0, out_ref.at[b_rows(0), HALF:N])
        for h in range(N_DEV - 1):
            recv_wait(comm_f, sf_recv, h)
            out_ref[f_rows(h + 1), 0:HALF] += comm_f[h]
            if h < N_DEV - 2:
                fwd_send(h + 1, out_ref.at[f_rows(h + 1), 0:HALF])
            recv_wait(comm_b, sb_recv, h)
            out_ref[b_rows(h + 1), HALF:N] += comm_b[h]
            if h < N_DEV - 2:
                bwd_send(h + 1, out_ref.at[b_rows(h + 1), HALF:N])

        AG = N_DEV - 1
        fwd_send(AG, out_ref.at[f_rows(N_DEV - 1), 0:HALF])
        bwd_send(AG, out_ref.at[b_rows(N_DEV - 1), HALF:N])
        for h in range(N_DEV - 1):
            recv_wait(comm_f, sf_recv, AG + h)
            if h < N_DEV - 2:
                fwd_send(AG + h + 1, comm_f.at[AG + h])
            out_ref[f_rows(h), 0:HALF] = comm_f[AG + h]
            recv_wait(comm_b, sb_recv, AG + h)
            if h < N_DEV - 2:
                bwd_send(AG + h + 1, comm_b.at[AG + h])
            out_ref[b_rows(h), HALF:N] = comm_b[AG + h]

        for s in range(N_HOPS):
            for comm, ssem, rsem, dev in (
                (comm_f, sf_send, sf_recv, right),
                (comm_b, sb_send, sb_recv, left),
            ):
                rdma = pltpu.make_async_remote_copy(
                    src_ref=comm.at[s], dst_ref=comm.at[s],
                    send_sem=ssem.at[s], recv_sem=rsem.at[s],
                    device_id=(dev,), device_id_type=pl.DeviceIdType.MESH,
                )
                rdma.wait_send()

    return pl.pallas_call(
        body,
        out_shape=jax.ShapeDtypeStruct((M, N), jnp.float32),
        in_specs=[pl.BlockSpec(memory_space=pltpu.VMEM)],
        out_specs=pl.BlockSpec(memory_space=pltpu.VMEM),
        scratch_shapes=[
            pltpu.VMEM((N_HOPS, CH, HALF), jnp.float32),
            pltpu.VMEM((N_HOPS, CH, HALF), jnp.float32),
            pltpu.SemaphoreType.DMA((N_HOPS,)),
            pltpu.SemaphoreType.DMA((N_HOPS,)),
            pltpu.SemaphoreType.DMA((N_HOPS,)),
            pltpu.SemaphoreType.DMA((N_HOPS,)),
        ],
        compiler_params=pltpu.CompilerParams(collective_id=0),
    )(x)


# device time: 123257 ns/iter; 1.9102x vs baseline; 1.2510x over previous
import jax
import jax.numpy as jnp
from jax import lax
from jax.experimental import pallas as pl
from jax.experimental.pallas import tpu as pltpu

N_DEV = 16
M = 2048
N = 1024
CH = M // N_DEV
HALF = N // 2
S = 4
CHS = CH // S
N_HOPS = 2 * (N_DEV - 1)


def kernel(x):
    def body(x_ref, out_ref, comm_f, comm_b, sf_send, sf_recv, sb_send, sb_recv):
        me = lax.axis_index("i")
        left = (me - 1) % N_DEV
        right = (me + 1) % N_DEV

        def fwd_send(h, s, src):
            pltpu.make_async_remote_copy(
                src_ref=src, dst_ref=comm_f.at[h, s],
                send_sem=sf_send.at[h, s], recv_sem=sf_recv.at[h, s],
                device_id=(right,), device_id_type=pl.DeviceIdType.MESH,
            ).start()

        def bwd_send(h, s, src):
            pltpu.make_async_remote_copy(
                src_ref=src, dst_ref=comm_b.at[h, s],
                send_sem=sb_send.at[h, s], recv_sem=sb_recv.at[h, s],
                device_id=(left,), device_id_type=pl.DeviceIdType.MESH,
            ).start()

        def recv_wait(comm, recv_sems, h, s):
            pltpu.make_async_remote_copy(
                src_ref=comm.at[h, s], dst_ref=comm.at[h, s],
                send_sem=sf_send.at[h, s], recv_sem=recv_sems.at[h, s],
                device_id=(right,), device_id_type=pl.DeviceIdType.MESH,
            ).wait_recv()

        barrier_sem = pltpu.get_barrier_semaphore()
        for nbr in (left, right):
            pl.semaphore_signal(
                barrier_sem, inc=1,
                device_id=(nbr,), device_id_type=pl.DeviceIdType.MESH,
            )
        pl.semaphore_wait(barrier_sem, 2)

        out_ref[:, :] = x_ref[0]

        def f_sub(h, s):
            return pl.ds(((me - h) % N_DEV) * CH + s * CHS, CHS)

        def b_sub(h, s):
            return pl.ds(((me + h) % N_DEV) * CH + s * CHS, CHS)

        for s in range(S):
            fwd_send(0, s, out_ref.at[f_sub(0, s), 0:HALF])
            bwd_send(0, s, out_ref.at[b_sub(0, s), HALF:N])
        for h in range(N_DEV - 1):
            for s in range(S):
                recv_wait(comm_f, sf_recv, h, s)
                out_ref[f_sub(h + 1, s), 0:HALF] += comm_f[h, s]
                if h < N_DEV - 2:
                    fwd_send(h + 1, s, out_ref.at[f_sub(h + 1, s), 0:HALF])
                recv_wait(comm_b, sb_recv, h, s)
                out_ref[b_sub(h + 1, s), HALF:N] += comm_b[h, s]
                if h < N_DEV - 2:
                    bwd_send(h + 1, s, out_ref.at[b_sub(h + 1, s), HALF:N])

        AG = N_DEV - 1
        for s in range(S):
            fwd_send(AG, s, out_ref.at[f_sub(AG, s), 0:HALF])
            bwd_send(AG, s, out_ref.at[b_sub(AG, s), HALF:N])
        for h in range(N_DEV - 1):
            for s in range(S):
                recv_wait(comm_f, sf_recv, AG + h, s)
                if h < N_DEV - 2:
                    fwd_send(AG + h + 1, s, comm_f.at[AG + h, s])
                out_ref[f_sub(h, s), 0:HALF] = comm_f[AG + h, s]
                recv_wait(comm_b, sb_recv, AG + h, s)
                if h < N_DEV - 2:
                    bwd_send(AG + h + 1, s, comm_b.at[AG + h, s])
                out_ref[b_sub(h, s), HALF:N] = comm_b[AG + h, s]

        for h in range(N_HOPS):
            for s in range(S):
                for comm, ssem, rsem, dev in (
                    (comm_f, sf_send, sf_recv, right),
                    (comm_b, sb_send, sb_recv, left),
                ):
                    pltpu.make_async_remote_copy(
                        src_ref=comm.at[h, s], dst_ref=comm.at[h, s],
                        send_sem=ssem.at[h, s], recv_sem=rsem.at[h, s],
                        device_id=(dev,), device_id_type=pl.DeviceIdType.MESH,
                    ).wait_send()

    return pl.pallas_call(
        body,
        out_shape=jax.ShapeDtypeStruct((M, N), jnp.float32),
        in_specs=[pl.BlockSpec(memory_space=pltpu.VMEM)],
        out_specs=pl.BlockSpec(memory_space=pltpu.VMEM),
        scratch_shapes=[
            pltpu.VMEM((N_HOPS, S, CHS, HALF), jnp.float32),
            pltpu.VMEM((N_HOPS, S, CHS, HALF), jnp.float32),
            pltpu.SemaphoreType.DMA((N_HOPS, S)),
            pltpu.SemaphoreType.DMA((N_HOPS, S)),
            pltpu.SemaphoreType.DMA((N_HOPS, S)),
            pltpu.SemaphoreType.DMA((N_HOPS, S)),
        ],
        compiler_params=pltpu.CompilerParams(collective_id=0),
    )(x)
